# baseline (speedup 1.0000x reference)
import functools

import jax
import jax.numpy as jnp
from jax import lax
from jax.experimental import pallas as pl
from jax.experimental.pallas import tpu as pltpu
from jax.experimental.pallas import tpu_sc as plsc

B = 4096
L = 200
H = 768
C = 1000
NW = 32
SAMPLES_PER_W = B // NW
CHUNK = 40
N_CHUNKS = L // CHUNK
HL = H // 16


def _pool_body(table_hbm, idx_hbm, out_hbm, idx_v, rows_v, acc_v, sem):
    wid = lax.axis_index("s") * 2 + lax.axis_index("c")
    base = wid * SAMPLES_PER_W

    def sample_body(s, carry):
        g = base + s
        pltpu.sync_copy(idx_hbm.at[pl.ds(g * L, L)], idx_v)
        zero = jnp.zeros((16,), jnp.float32)
        for c in range(HL):
            acc_v[pl.ds(16 * c, 16)] = zero
        for k in range(N_CHUNKS):
            pltpu.async_copy(
                table_hbm.at[idx_v.at[pl.ds(k * CHUNK, CHUNK)]], rows_v, sem
            ).wait()

            def row_body(j, carry2):
                for c in range(HL):
                    plsc.addupdate(
                        acc_v.at[pl.ds(16 * c, 16)],
                        rows_v[j, pl.ds(16 * c, 16)],
                    )
                return carry2

            lax.fori_loop(0, CHUNK, row_body, 0, unroll=False)
        pltpu.sync_copy(acc_v, out_hbm.at[g])
        return carry

    lax.fori_loop(0, SAMPLES_PER_W, sample_body, 0, unroll=False)


@jax.jit
def _pool(table, idx_flat):
    mesh = plsc.VectorSubcoreMesh(core_axis_name="c", subcore_axis_name="s")
    return pl.kernel(
        _pool_body,
        out_type=jax.ShapeDtypeStruct((B, H), jnp.float32),
        mesh=mesh,
        scratch_types=[
            pltpu.VMEM((L,), jnp.int32),
            pltpu.VMEM((CHUNK, H), jnp.float32),
            pltpu.VMEM((H,), jnp.float32),
            pltpu.SemaphoreType.DMA,
        ],
    )(table, idx_flat)


def _mm_body(x_ref, w_ref, b_ref, o_ref):
    acc = jnp.dot(x_ref[...], w_ref[...], preferred_element_type=jnp.float32)
    o_ref[...] = acc * (1.0 / L) + b_ref[...]


@jax.jit
def _matmul(x, W, b2):
    BB = 1024
    return pl.pallas_call(
        _mm_body,
        grid=(B // BB,),
        in_specs=[
            pl.BlockSpec((BB, H), lambda i: (i, 0)),
            pl.BlockSpec((H, C), lambda i: (0, 0)),
            pl.BlockSpec((1, C), lambda i: (0, 0)),
        ],
        out_specs=pl.BlockSpec((BB, C), lambda i: (i, 0)),
        out_shape=jax.ShapeDtypeStruct((B, C), jnp.float32),
    )(x, W, b2)


def kernel(input_vectors, table, W, b):
    idx_flat = input_vectors.reshape(B * L).astype(jnp.int32)
    pooled_sum = _pool(table, idx_flat)
    return _matmul(pooled_sum, W, b.reshape(1, C))


# speedup vs baseline: 3.7278x; 3.7278x over previous
import functools

import jax
import jax.numpy as jnp
from jax import lax
from jax.experimental import pallas as pl
from jax.experimental.pallas import tpu as pltpu
from jax.experimental.pallas import tpu_sc as plsc

B = 4096
L = 200
H = 768
C = 1000
NW = 32
SAMPLES_PER_W = B // NW
CHUNK = 40
N_CHUNKS = L // CHUNK
HL = H // 16


def _pool_body(table_hbm, idx_hbm, out_hbm, idx_v, rows_a, rows_b, acc_v,
               sem_a, sem_b):
    wid = lax.axis_index("s") * 2 + lax.axis_index("c")
    base = wid * SAMPLES_PER_W
    pltpu.sync_copy(idx_hbm.at[pl.ds(base * L, SAMPLES_PER_W * L)], idx_v)

    bufs = (rows_a, rows_b)
    sems = (sem_a, sem_b)

    def gather(s, k, buf, sem):
        return pltpu.async_copy(
            table_hbm.at[idx_v.at[pl.ds(s * L + k * CHUNK, CHUNK)]], buf, sem
        )

    def accumulate(buf, acc):
        def row_body(j, a):
            return tuple(a[c] + buf[j, pl.ds(16 * c, 16)] for c in range(HL))

        return lax.fori_loop(0, CHUNK, row_body, acc)

    def sample_body(s, carry):
        acc = tuple(jnp.zeros((16,), jnp.float32) for _ in range(HL))
        h = gather(s, 0, bufs[0], sems[0])
        for k in range(N_CHUNKS):
            h.wait()
            if k + 1 < N_CHUNKS:
                h = gather(s, k + 1, bufs[(k + 1) % 2], sems[(k + 1) % 2])
            acc = accumulate(bufs[k % 2], acc)
        for c in range(HL):
            acc_v[pl.ds(16 * c, 16)] = acc[c]
        pltpu.sync_copy(acc_v, out_hbm.at[base + s])
        return carry

    lax.fori_loop(0, SAMPLES_PER_W, sample_body, 0, unroll=False)


@jax.jit
def _pool(table, idx_flat):
    mesh = plsc.VectorSubcoreMesh(core_axis_name="c", subcore_axis_name="s")
    return pl.kernel(
        _pool_body,
        out_type=jax.ShapeDtypeStruct((B, H), jnp.float32),
        mesh=mesh,
        scratch_types=[
            pltpu.VMEM((SAMPLES_PER_W * L,), jnp.int32),
            pltpu.VMEM((CHUNK, H), jnp.float32),
            pltpu.VMEM((CHUNK, H), jnp.float32),
            pltpu.VMEM((H,), jnp.float32),
            pltpu.SemaphoreType.DMA,
            pltpu.SemaphoreType.DMA,
        ],
    )(table, idx_flat)


def _mm_body(x_ref, w_ref, b_ref, o_ref):
    acc = jnp.dot(x_ref[...], w_ref[...], preferred_element_type=jnp.float32)
    o_ref[...] = acc * (1.0 / L) + b_ref[...]


@jax.jit
def _matmul(x, W, b2):
    BB = 1024
    return pl.pallas_call(
        _mm_body,
        grid=(B // BB,),
        in_specs=[
            pl.BlockSpec((BB, H), lambda i: (i, 0)),
            pl.BlockSpec((H, C), lambda i: (0, 0)),
            pl.BlockSpec((1, C), lambda i: (0, 0)),
        ],
        out_specs=pl.BlockSpec((BB, C), lambda i: (i, 0)),
        out_shape=jax.ShapeDtypeStruct((B, C), jnp.float32),
    )(x, W, b2)


def kernel(input_vectors, table, W, b):
    idx_flat = input_vectors.reshape(B * L).astype(jnp.int32)
    pooled_sum = _pool(table, idx_flat)
    return _matmul(pooled_sum, W, b.reshape(1, C))
